# baseline (speedup 1.0000x reference)
import functools

import jax
import jax.numpy as jnp
from jax import lax
from jax.experimental import pallas as pl
from jax.experimental.pallas import tpu as pltpu
from jax.experimental.pallas import tpu_sc as plsc

N = 10000
D = 512
NREL = 3
E = 40000

NPAD = 10240
RT = NPAD // 16
DC = 128
NCHUNK = D // DC
EPAD = 40960
ET = EPAD // 16
EB = 128
NB = ET // EB
ZR = 32

_mesh = plsc.VectorSubcoreMesh(core_axis_name="c", subcore_axis_name="s")


def _sc_body(x0, x1, x2, x3, src_ref, dst_ref, agg_ref, deg_ref,
             acc, dega, sidx, didx, rows_a, rows_b, zbuf, ones,
             sem_ga, sem_gb, sem_sa, sem_sb, sem_d, sem_z):
    c = lax.axis_index("c")
    s = lax.axis_index("s")
    row0 = s * RT

    zv = jnp.zeros((16,), jnp.float32)
    ov = jnp.ones((16,), jnp.float32)

    def _init(i, carry):
        for j in range(DC // 16):
            zbuf[i, pl.ds(j * 16, 16)] = zv
        return carry

    lax.fori_loop(0, ZR, _init, 0)
    for j in range(EB // 16):
        ones[pl.ds(j * 16, 16)] = ov

    xcs = [x0, x1, x2, x3]
    for fc in range(NCHUNK // 2):
        for cc in range(2):
            chunk = fc * 2 + cc
            xc = xcs[chunk]
            do_deg = (chunk == 0)

            @pl.when(c == cc)
            def _chunk_pass(xc=xc, chunk=chunk, do_deg=do_deg):
                def _gather(b, buf, sem):
                    pltpu.async_copy(xc.at[sidx.at[b]], buf, sem)

                def _gwait(buf, sem):
                    pltpu.make_async_copy(xc.at[sidx.at[0]], buf, sem).wait()

                def _scat(b, buf, sem):
                    pltpu.async_copy(buf, acc.at[didx.at[b]], sem, add=True)

                def _swait(buf, sem):
                    pltpu.make_async_copy(buf, acc.at[didx.at[0]], sem).wait()

                def _dscat(b):
                    pltpu.async_copy(ones, dega.at[didx.at[b]], sem_d,
                                     add=True)

                def _dwait():
                    pltpu.make_async_copy(ones, dega.at[didx.at[0]],
                                          sem_d).wait()

                for r in range(NREL):
                    pltpu.sync_copy(src_ref.at[r, s], sidx)
                    pltpu.sync_copy(dst_ref.at[r, s], didx)

                    def _zero(k, carry):
                        pltpu.async_copy(
                            zbuf, acc.at[pl.ds(row0 + k * ZR, ZR)], sem_z)
                        return carry

                    lax.fori_loop(0, RT // ZR, _zero, 0)
                    if do_deg:
                        def _zerod(k, carry):
                            pltpu.async_copy(
                                zbuf.at[0],
                                dega.at[pl.ds(row0 + k * DC, DC)], sem_z)
                            return carry

                        lax.fori_loop(0, RT // DC, _zerod, 0)

                    def _zwait(k, carry):
                        pltpu.make_async_copy(
                            zbuf, acc.at[pl.ds(row0, ZR)], sem_z).wait()
                        return carry

                    lax.fori_loop(0, RT // ZR, _zwait, 0)
                    if do_deg:
                        def _zwaitd(k, carry):
                            pltpu.make_async_copy(
                                zbuf.at[0], dega.at[pl.ds(row0, DC)],
                                sem_z).wait()
                            return carry

                        lax.fori_loop(0, RT // DC, _zwaitd, 0)
                    plsc.subcore_barrier()

                    _gather(0, rows_a, sem_ga)
                    _gather(1, rows_b, sem_gb)

                    def _pair(h, carry):
                        b0 = 2 * h
                        _gwait(rows_a, sem_ga)
                        _scat(b0, rows_a, sem_sa)
                        if do_deg:
                            _dscat(b0)
                        _gwait(rows_b, sem_gb)
                        _scat(b0 + 1, rows_b, sem_sb)
                        if do_deg:
                            _dscat(b0 + 1)
                        nxt = jnp.where(b0 + 2 < NB, b0 + 2, 0)
                        _swait(rows_a, sem_sa)
                        _gather(nxt, rows_a, sem_ga)
                        nxt2 = jnp.where(b0 + 3 < NB, b0 + 3, 0)
                        _swait(rows_b, sem_sb)
                        _gather(nxt2, rows_b, sem_gb)
                        return carry

                    lax.fori_loop(0, NB // 2, _pair, 0)
                    _gwait(rows_a, sem_ga)
                    _gwait(rows_b, sem_gb)
                    if do_deg:
                        def _ddrain(k, carry):
                            _dwait()
                            return carry

                        lax.fori_loop(0, NB, _ddrain, 0)
                    plsc.subcore_barrier()

                    pltpu.sync_copy(
                        acc.at[pl.ds(row0, RT)],
                        agg_ref.at[r, pl.ds(row0, RT), pl.ds(chunk * DC, DC)])
                    if do_deg:
                        pltpu.sync_copy(dega.at[pl.ds(row0, RT)],
                                        deg_ref.at[r, 0, pl.ds(row0, RT)])
                    plsc.subcore_barrier()


_sc_aggregate = functools.partial(
    pl.kernel,
    out_type=[
        jax.ShapeDtypeStruct((NREL, NPAD, D), jnp.float32),
        jax.ShapeDtypeStruct((NREL, 1, NPAD), jnp.float32),
    ],
    mesh=_mesh,
    scratch_types=[
        pltpu.VMEM_SHARED((NPAD, DC), jnp.float32),
        pltpu.VMEM_SHARED((NPAD,), jnp.float32),
        pltpu.VMEM((NB, EB), jnp.int32),
        pltpu.VMEM((NB, EB), jnp.int32),
        pltpu.VMEM((EB, DC), jnp.float32),
        pltpu.VMEM((EB, DC), jnp.float32),
        pltpu.VMEM((ZR, DC), jnp.float32),
        pltpu.VMEM((EB,), jnp.float32),
        pltpu.SemaphoreType.DMA,
        pltpu.SemaphoreType.DMA,
        pltpu.SemaphoreType.DMA,
        pltpu.SemaphoreType.DMA,
        pltpu.SemaphoreType.DMA,
        pltpu.SemaphoreType.DMA,
    ],
)(_sc_body)


ROWB = 400
NROWB = N // ROWB


def _tc_body(x_ref, agg_ref, deg_ref, w_ref, ws_ref, o_ref):
    acc = jnp.dot(x_ref[...], ws_ref[...], preferred_element_type=jnp.float32)
    for r in range(NREL):
        inv = 1.0 / jnp.maximum(deg_ref[0, r], 1.0)
        acc = acc + jnp.dot(agg_ref[r] * inv[:, None], w_ref[r],
                            preferred_element_type=jnp.float32)
    o_ref[...] = acc


def _tc_matmul(x, agg, deg3, W, W_self):
    return pl.pallas_call(
        _tc_body,
        grid=(NROWB,),
        in_specs=[
            pl.BlockSpec((ROWB, D), lambda i: (i, 0)),
            pl.BlockSpec((NREL, ROWB, D), lambda i: (0, i, 0)),
            pl.BlockSpec((1, NREL, ROWB), lambda i: (i, 0, 0)),
            pl.BlockSpec((NREL, D, D), lambda i: (0, 0, 0)),
            pl.BlockSpec((D, D), lambda i: (0, 0)),
        ],
        out_specs=pl.BlockSpec((ROWB, D), lambda i: (i, 0)),
        out_shape=jax.ShapeDtypeStruct((N, D), jnp.float32),
    )(x, agg, deg3, W, W_self)


def kernel(x, edge_index_r0, edge_index_r1, edge_index_r2, W, W_self):
    ei = jnp.stack([edge_index_r0, edge_index_r1, edge_index_r2]).astype(jnp.int32)
    src = jnp.concatenate(
        [ei[:, 0, :], jnp.zeros((NREL, EPAD - E), jnp.int32)],
        axis=1).reshape(NREL, 16, NB, EB)
    dst = jnp.concatenate(
        [ei[:, 1, :], jnp.full((NREL, EPAD - E), N, jnp.int32)],
        axis=1).reshape(NREL, 16, NB, EB)
    xcs = [x[:, k * DC:(k + 1) * DC] for k in range(NCHUNK)]
    agg, deg = _sc_aggregate(xcs[0], xcs[1], xcs[2], xcs[3], src, dst)
    deg3 = deg[:, 0, :N].reshape(NREL, NROWB, ROWB).transpose(1, 0, 2)
    return _tc_matmul(x, agg, deg3, W, W_self)


# speedup vs baseline: 1.0641x; 1.0641x over previous
import functools

import jax
import jax.numpy as jnp
from jax import lax
from jax.experimental import pallas as pl
from jax.experimental.pallas import tpu as pltpu
from jax.experimental.pallas import tpu_sc as plsc

N = 10000
D = 512
NREL = 3
E = 40000

NPAD = 10240
RT = NPAD // 16
DC = 128
NCHUNK = D // DC
EPAD = 40960
ET = EPAD // 16
EB = 128
NB = ET // EB
ZR = 32

_mesh = plsc.VectorSubcoreMesh(core_axis_name="c", subcore_axis_name="s")


def _sc_body(x0, x1, x2, x3, sd_ref, agg_ref, deg_ref,
             acc, dega, sdidx, rows_a, rows_b, zbuf, ones,
             sem_ga, sem_gb):
    c = lax.axis_index("c")
    s = lax.axis_index("s")
    row0 = s * RT

    zv = jnp.zeros((16,), jnp.float32)
    ov = jnp.ones((16,), jnp.float32)

    def _init(i, carry):
        for j in range(DC // 16):
            zbuf[i, pl.ds(j * 16, 16)] = zv
        return carry

    lax.fori_loop(0, ZR, _init, 0)
    for j in range(EB // 16):
        ones[pl.ds(j * 16, 16)] = ov

    xcs = [x0, x1, x2, x3]
    for fc in range(NCHUNK // 2):
        for cc in range(2):
            chunk = fc * 2 + cc
            xc = xcs[chunk]

            @pl.when(c == cc)
            def _chunk_pass(xc=xc, chunk=chunk):
                def _gather(b, buf, sem):
                    pltpu.async_copy(xc.at[sdidx.at[0, b]], buf, sem)

                def _gwait(buf, sem):
                    pltpu.make_async_copy(xc.at[sdidx.at[0, 0]], buf,
                                          sem).wait()

                for r in range(NREL):
                    do_deg = (fc == 0 and r % 2 == cc)
                    pltpu.sync_copy(sd_ref.at[r, s], sdidx)

                    def _zero(k, carry):
                        pltpu.sync_copy(zbuf, acc.at[pl.ds(row0 + k * ZR, ZR)])
                        return carry

                    lax.fori_loop(0, RT // ZR, _zero, 0)
                    if do_deg:
                        def _zerod(k, carry):
                            pltpu.sync_copy(
                                zbuf.at[0],
                                dega.at[pl.ds(row0 + k * DC, DC)])
                            return carry

                        lax.fori_loop(0, RT // DC, _zerod, 0)
                    plsc.subcore_barrier()

                    _gather(0, rows_a, sem_ga)

                    def _pair(h, carry):
                        b0 = 2 * h
                        _gather(b0 + 1, rows_b, sem_gb)
                        _gwait(rows_a, sem_ga)
                        pltpu.sync_copy(rows_a, acc.at[sdidx.at[1, b0]], add=True)
                        if do_deg:
                            pltpu.sync_copy(ones, dega.at[sdidx.at[1, b0]],
                                            add=True)
                        nxt = jnp.where(b0 + 2 < NB, b0 + 2, 0)
                        _gather(nxt, rows_a, sem_ga)
                        _gwait(rows_b, sem_gb)
                        pltpu.sync_copy(rows_b, acc.at[sdidx.at[1, b0 + 1]],
                                        add=True)
                        if do_deg:
                            pltpu.sync_copy(ones, dega.at[sdidx.at[1, b0 + 1]],
                                            add=True)
                        return carry

                    lax.fori_loop(0, NB // 2, _pair, 0)
                    _gwait(rows_a, sem_ga)
                    plsc.subcore_barrier()

                    pltpu.sync_copy(
                        acc.at[pl.ds(row0, RT)],
                        agg_ref.at[r, pl.ds(row0, RT), pl.ds(chunk * DC, DC)])
                    if do_deg:
                        pltpu.sync_copy(dega.at[pl.ds(row0, RT)],
                                        deg_ref.at[r, 0, pl.ds(row0, RT)])
                    plsc.subcore_barrier()


_sc_aggregate = functools.partial(
    pl.kernel,
    out_type=[
        jax.ShapeDtypeStruct((NREL, NPAD, D), jnp.float32),
        jax.ShapeDtypeStruct((NREL, 1, NPAD), jnp.float32),
    ],
    mesh=_mesh,
    scratch_types=[
        pltpu.VMEM_SHARED((NPAD, DC), jnp.float32),
        pltpu.VMEM_SHARED((NPAD,), jnp.float32),
        pltpu.VMEM((2, NB, EB), jnp.int32),
        pltpu.VMEM((EB, DC), jnp.float32),
        pltpu.VMEM((EB, DC), jnp.float32),
        pltpu.VMEM((ZR, DC), jnp.float32),
        pltpu.VMEM((EB,), jnp.float32),
        pltpu.SemaphoreType.DMA,
        pltpu.SemaphoreType.DMA,
    ],
)(_sc_body)


ROWB = 400
NROWB = N // ROWB


def _tc_self_body(x_ref, ws_ref, o_ref):
    o_ref[...] = jnp.dot(x_ref[...], ws_ref[...],
                         preferred_element_type=jnp.float32)


def _tc_self(x, W_self):
    return pl.pallas_call(
        _tc_self_body,
        grid=(NROWB,),
        in_specs=[
            pl.BlockSpec((ROWB, D), lambda i: (i, 0)),
            pl.BlockSpec((D, D), lambda i: (0, 0)),
        ],
        out_specs=pl.BlockSpec((ROWB, D), lambda i: (i, 0)),
        out_shape=jax.ShapeDtypeStruct((N, D), jnp.float32),
    )(x, W_self)


def _tc_body(h_ref, agg_ref, deg_ref, w_ref, o_ref):
    acc = h_ref[...]
    for r in range(NREL):
        inv = 1.0 / jnp.maximum(deg_ref[0, r], 1.0)
        acc = acc + jnp.dot(agg_ref[r] * inv[:, None], w_ref[r],
                            preferred_element_type=jnp.float32)
    o_ref[...] = acc


def _tc_matmul(h_self, agg, deg3, W):
    return pl.pallas_call(
        _tc_body,
        grid=(NROWB,),
        in_specs=[
            pl.BlockSpec((ROWB, D), lambda i: (i, 0)),
            pl.BlockSpec((NREL, ROWB, D), lambda i: (0, i, 0)),
            pl.BlockSpec((1, NREL, ROWB), lambda i: (i, 0, 0)),
            pl.BlockSpec((NREL, D, D), lambda i: (0, 0, 0)),
        ],
        out_specs=pl.BlockSpec((ROWB, D), lambda i: (i, 0)),
        out_shape=jax.ShapeDtypeStruct((N, D), jnp.float32),
    )(h_self, agg, deg3, W)


def kernel(x, edge_index_r0, edge_index_r1, edge_index_r2, W, W_self):
    ei = jnp.stack([edge_index_r0, edge_index_r1, edge_index_r2]).astype(jnp.int32)
    src = jnp.concatenate(
        [ei[:, 0, :], jnp.zeros((NREL, EPAD - E), jnp.int32)],
        axis=1).reshape(NREL, 16, NB, EB)
    dst = jnp.concatenate(
        [ei[:, 1, :], jnp.full((NREL, EPAD - E), N, jnp.int32)],
        axis=1).reshape(NREL, 16, NB, EB)
    sd = jnp.stack([src, dst], axis=2)
    xcs = [x[:, k * DC:(k + 1) * DC] for k in range(NCHUNK)]
    h_self = _tc_self(x, W_self)
    agg, deg = _sc_aggregate(xcs[0], xcs[1], xcs[2], xcs[3], sd)
    deg3 = deg[:, 0, :N].reshape(NREL, NROWB, ROWB).transpose(1, 0, 2)
    return _tc_matmul(h_self, agg, deg3, W)


# speedup vs baseline: 1.0783x; 1.0134x over previous
import functools

import jax
import jax.numpy as jnp
from jax import lax
from jax.experimental import pallas as pl
from jax.experimental.pallas import tpu as pltpu
from jax.experimental.pallas import tpu_sc as plsc

N = 10000
D = 512
NREL = 3
E = 40000

NPAD = 10240
RT = NPAD // 16
DC = 128
NCHUNK = D // DC
EPAD = 40960
ET = EPAD // 16
EB = 128
NB = ET // EB
ZR = 32

_mesh = plsc.VectorSubcoreMesh(core_axis_name="c", subcore_axis_name="s")


def _sc_body(x0, x1, x2, x3, sd_ref, agg_ref, deg_ref,
             acc, dega, sdidx, rows_a, rows_b, zbuf, ones,
             sem_ga, sem_gb, sem_z):
    c = lax.axis_index("c")
    s = lax.axis_index("s")
    row0 = s * RT

    zv = jnp.zeros((16,), jnp.float32)
    ov = jnp.ones((16,), jnp.float32)

    def _init(i, carry):
        for j in range(DC // 16):
            zbuf[i, pl.ds(j * 16, 16)] = zv
        return carry

    lax.fori_loop(0, ZR, _init, 0)
    for j in range(EB // 16):
        ones[pl.ds(j * 16, 16)] = ov

    xcs = [x0, x1, x2, x3]
    for fc in range(NCHUNK // 2):
        for cc in range(2):
            chunk = fc * 2 + cc
            xc = xcs[chunk]

            @pl.when(c == cc)
            def _chunk_pass(xc=xc, chunk=chunk):
                def _gather(b, buf, sem):
                    pltpu.async_copy(xc.at[sdidx.at[0, b]], buf, sem)

                def _gwait(buf, sem):
                    pltpu.make_async_copy(xc.at[sdidx.at[0, 0]], buf,
                                          sem).wait()

                def _zero_fire(dd):
                    def _zf(k, carry):
                        pltpu.async_copy(
                            zbuf, acc.at[pl.ds(row0 + k * ZR, ZR)], sem_z)
                        return carry

                    lax.fori_loop(0, RT // ZR, _zf, 0)
                    if dd:
                        def _zfd(k, carry):
                            pltpu.async_copy(
                                zbuf.at[0],
                                dega.at[pl.ds(row0 + k * DC, DC)], sem_z)
                            return carry

                        lax.fori_loop(0, RT // DC, _zfd, 0)

                    def _zw(k, carry):
                        pltpu.make_async_copy(
                            zbuf, acc.at[pl.ds(row0, ZR)], sem_z).wait()
                        return carry

                    lax.fori_loop(0, RT // ZR, _zw, 0)
                    if dd:
                        def _zwd(k, carry):
                            pltpu.make_async_copy(
                                zbuf.at[0], dega.at[pl.ds(row0, DC)],
                                sem_z).wait()
                            return carry

                        lax.fori_loop(0, RT // DC, _zwd, 0)

                def _dd(r):
                    return fc == 0 and r % 2 == cc

                _zero_fire(_dd(0))
                for r in range(NREL):
                    do_deg = _dd(r)
                    plsc.subcore_barrier()

                    pltpu.sync_copy(sd_ref.at[r, s], sdidx)

                    _gather(0, rows_a, sem_ga)

                    def _pair(h, carry):
                        b0 = 2 * h
                        _gather(b0 + 1, rows_b, sem_gb)
                        _gwait(rows_a, sem_ga)
                        pltpu.sync_copy(rows_a, acc.at[sdidx.at[1, b0]], add=True)
                        if do_deg:
                            pltpu.sync_copy(ones, dega.at[sdidx.at[1, b0]],
                                            add=True)
                        nxt = jnp.where(b0 + 2 < NB, b0 + 2, 0)
                        _gather(nxt, rows_a, sem_ga)
                        _gwait(rows_b, sem_gb)
                        pltpu.sync_copy(rows_b, acc.at[sdidx.at[1, b0 + 1]],
                                        add=True)
                        if do_deg:
                            pltpu.sync_copy(ones, dega.at[sdidx.at[1, b0 + 1]],
                                            add=True)
                        return carry

                    lax.fori_loop(0, NB // 2, _pair, 0)
                    _gwait(rows_a, sem_ga)
                    plsc.subcore_barrier()

                    pltpu.sync_copy(
                        acc.at[pl.ds(row0, RT)],
                        agg_ref.at[r, pl.ds(row0, RT), pl.ds(chunk * DC, DC)])
                    if do_deg:
                        pltpu.sync_copy(dega.at[pl.ds(row0, RT)],
                                        deg_ref.at[r, 0, pl.ds(row0, RT)])
                    if r < NREL - 1:
                        _zero_fire(_dd(r + 1))


_sc_aggregate = functools.partial(
    pl.kernel,
    out_type=[
        jax.ShapeDtypeStruct((NREL, NPAD, D), jnp.float32),
        jax.ShapeDtypeStruct((NREL, 1, NPAD), jnp.float32),
    ],
    mesh=_mesh,
    scratch_types=[
        pltpu.VMEM_SHARED((NPAD, DC), jnp.float32),
        pltpu.VMEM_SHARED((NPAD,), jnp.float32),
        pltpu.VMEM((2, NB, EB), jnp.int32),
        pltpu.VMEM((EB, DC), jnp.float32),
        pltpu.VMEM((EB, DC), jnp.float32),
        pltpu.VMEM((ZR, DC), jnp.float32),
        pltpu.VMEM((EB,), jnp.float32),
        pltpu.SemaphoreType.DMA,
        pltpu.SemaphoreType.DMA,
        pltpu.SemaphoreType.DMA,
    ],
)(_sc_body)


ROWB = 400
NROWB = N // ROWB


def _tc_self_body(x_ref, ws_ref, o_ref):
    o_ref[...] = jnp.dot(x_ref[...], ws_ref[...],
                         preferred_element_type=jnp.float32)


def _tc_self(x, W_self):
    return pl.pallas_call(
        _tc_self_body,
        grid=(NROWB,),
        in_specs=[
            pl.BlockSpec((ROWB, D), lambda i: (i, 0)),
            pl.BlockSpec((D, D), lambda i: (0, 0)),
        ],
        out_specs=pl.BlockSpec((ROWB, D), lambda i: (i, 0)),
        out_shape=jax.ShapeDtypeStruct((N, D), jnp.float32),
    )(x, W_self)


def _tc_body(h_ref, agg_ref, deg_ref, w_ref, o_ref):
    acc = h_ref[...]
    for r in range(NREL):
        inv = 1.0 / jnp.maximum(deg_ref[0, r], 1.0)
        acc = acc + jnp.dot(agg_ref[r] * inv[:, None], w_ref[r],
                            preferred_element_type=jnp.float32)
    o_ref[...] = acc


def _tc_matmul(h_self, agg, deg3, W):
    return pl.pallas_call(
        _tc_body,
        grid=(NROWB,),
        in_specs=[
            pl.BlockSpec((ROWB, D), lambda i: (i, 0)),
            pl.BlockSpec((NREL, ROWB, D), lambda i: (0, i, 0)),
            pl.BlockSpec((1, NREL, ROWB), lambda i: (i, 0, 0)),
            pl.BlockSpec((NREL, D, D), lambda i: (0, 0, 0)),
        ],
        out_specs=pl.BlockSpec((ROWB, D), lambda i: (i, 0)),
        out_shape=jax.ShapeDtypeStruct((N, D), jnp.float32),
    )(h_self, agg, deg3, W)


def kernel(x, edge_index_r0, edge_index_r1, edge_index_r2, W, W_self):
    ei = jnp.stack([edge_index_r0, edge_index_r1, edge_index_r2]).astype(jnp.int32)
    src = jnp.concatenate(
        [ei[:, 0, :], jnp.zeros((NREL, EPAD - E), jnp.int32)],
        axis=1).reshape(NREL, 16, NB, EB)
    dst = jnp.concatenate(
        [ei[:, 1, :], jnp.full((NREL, EPAD - E), N, jnp.int32)],
        axis=1).reshape(NREL, 16, NB, EB)
    sd = jnp.stack([src, dst], axis=2)
    xcs = [x[:, k * DC:(k + 1) * DC] for k in range(NCHUNK)]
    h_self = _tc_self(x, W_self)
    agg, deg = _sc_aggregate(xcs[0], xcs[1], xcs[2], xcs[3], sd)
    deg3 = deg[:, 0, :N].reshape(NREL, NROWB, ROWB).transpose(1, 0, 2)
    return _tc_matmul(h_self, agg, deg3, W)
